# baseline (speedup 1.0000x reference)
import functools

import jax
import jax.numpy as jnp
from jax import lax
from jax.experimental import pallas as pl
from jax.experimental.pallas import tpu as pltpu
from jax.experimental.pallas import tpu_sc as plsc

NUM_USERS = 1000000
BATCH = 16384
EMBED_DIM = 64
PANEL_W = 128
NUM_PANELS = NUM_USERS // PANEL_W
TAIL_BASE = NUM_PANELS * PANEL_W
TAIL_W = NUM_USERS - TAIL_BASE
NUM_CORES = 2
NUM_SUBCORES = 16
NUM_WORKERS = NUM_CORES * NUM_SUBCORES
ROWS_PER_WORKER = BATCH // NUM_WORKERS
NUM_Q = 4
QBATCH = BATCH // NUM_Q
QROWS = QBATCH // NUM_WORKERS
LANES = 16
ST_LEN = QROWS + LANES
GROUPS = ROWS_PER_WORKER // LANES
CHUNK = 32

_mesh = plsc.VectorSubcoreMesh(core_axis_name="c", subcore_axis_name="s")


def _sread(ref, j):
    base = lax.shift_right_logical(j, 4) * LANES
    v = ref[pl.ds(base, LANES)]
    lane = jnp.full((LANES,), j & (LANES - 1), jnp.int32)
    return v.at[lane].get(mode="promise_in_bounds")[0]


@functools.partial(
    pl.kernel,
    out_type=(
        jax.ShapeDtypeStruct((NUM_WORKERS, QROWS, EMBED_DIM), jnp.float32),
        jax.ShapeDtypeStruct((NUM_WORKERS, QROWS, EMBED_DIM), jnp.float32),
    ),
    mesh=_mesh,
    compiler_params=pltpu.CompilerParams(needs_layout_passes=False),
    scratch_types=[
        pltpu.VMEM((QROWS,), jnp.int32),
        pltpu.VMEM((QROWS,), jnp.int32),
        pltpu.VMEM((ST_LEN,), jnp.int32),
        pltpu.VMEM((2 * LANES,), jnp.int32),
        pltpu.VMEM((EMBED_DIM, PANEL_W), jnp.float32),
        pltpu.VMEM((TAIL_W, EMBED_DIM), jnp.float32),
        pltpu.VMEM((QROWS, EMBED_DIM), jnp.float32),
    ],
)
def _gather_kernel(su_hbm, ups_hbm, ust_hbm, uns_hbm,
                   sv_hbm, vps_hbm, vst_hbm, vns_hbm,
                   utT_hbm, itT_hbm, utail_hbm, itail_hbm,
                   stg_u_hbm, stg_v_hbm,
                   sid_v, ps_v, st_v, ns_v, panel, tailb, bigbuf):
    wid = lax.axis_index("s") * NUM_CORES + lax.axis_index("c")

    def do_table(ids_hbm, ps_hbm, sta_hbm, ns_hbm, tbl_hbm, tail_hbm,
                 stg_hbm):
        pltpu.sync_copy(ids_hbm.at[wid], sid_v)
        pltpu.sync_copy(ps_hbm.at[wid], ps_v)
        pltpu.sync_copy(sta_hbm.at[wid], st_v)
        pltpu.sync_copy(ns_hbm, ns_v)
        pltpu.sync_copy(tail_hbm, tailb)
        nslots = _sread(ns_v, wid)

        def id_body(j, c2):
            id_ = _sread(sid_v, j)
            is_tail = id_ >= TAIL_BASE
            col = jnp.full((LANES,), id_ & (PANEL_W - 1), jnp.int32)
            tr = jnp.full(
                (LANES,), jnp.where(is_tail, id_ - TAIL_BASE, 0), jnp.int32)
            for c in range(EMBED_DIM // LANES):
                dvec = jnp.full((LANES,), c * LANES, jnp.int32) \
                    + lax.iota(jnp.int32, LANES)
                u_p = plsc.load_gather(panel, [dvec, col])
                u_t = plsc.load_gather(tailb, [tr, dvec])
                bigbuf[j, pl.ds(c * LANES, LANES)] = \
                    jnp.where(is_tail, u_t, u_p)
            return c2

        def slot_body(s, carry):
            pnl = _sread(ps_v, s)
            start = _sread(st_v, s)
            end = _sread(st_v, s + 1)
            off = pl.multiple_of(pnl * PANEL_W, PANEL_W)
            pltpu.sync_copy(tbl_hbm.at[:, pl.ds(off, PANEL_W)], panel)
            lax.fori_loop(start, end, id_body, 0)
            return carry

        lax.fori_loop(0, nslots, slot_body, 0)
        pltpu.sync_copy(bigbuf, stg_hbm.at[wid])

    do_table(su_hbm, ups_hbm, ust_hbm, uns_hbm, utT_hbm, utail_hbm,
             stg_u_hbm)
    do_table(sv_hbm, vps_hbm, vst_hbm, vns_hbm, itT_hbm, itail_hbm,
             stg_v_hbm)


@functools.partial(
    pl.kernel,
    out_type=jax.ShapeDtypeStruct((NUM_WORKERS, ROWS_PER_WORKER), jnp.float32),
    mesh=_mesh,
    compiler_params=pltpu.CompilerParams(needs_layout_passes=False),
    scratch_types=[
        pltpu.VMEM((ROWS_PER_WORKER,), jnp.int32),
        pltpu.VMEM((ROWS_PER_WORKER,), jnp.int32),
        pltpu.VMEM((CHUNK, EMBED_DIM), jnp.float32),
        pltpu.VMEM((CHUNK, EMBED_DIM), jnp.float32),
        pltpu.VMEM((ROWS_PER_WORKER,), jnp.float32),
        pltpu.SemaphoreType.DMA,
        pltpu.SemaphoreType.DMA,
    ],
)
def _dot_kernel(ui_hbm, vi_hbm, stg_u_hbm, stg_v_hbm, out_hbm,
                ui_v, vi_v, rows_u, rows_v, out_vals, sem_u, sem_v):
    wid = lax.axis_index("s") * NUM_CORES + lax.axis_index("c")

    pltpu.sync_copy(ui_hbm.at[wid], ui_v)
    pltpu.sync_copy(vi_hbm.at[wid], vi_v)

    def chunk_body(ch, carry):
        base = ch * CHUNK
        copies = []
        for g in range(CHUNK // LANES):
            uvec = ui_v[pl.ds(base + g * LANES, LANES)]
            ivec = vi_v[pl.ds(base + g * LANES, LANES)]
            for i in range(LANES):
                k = g * LANES + i
                copies.append(pltpu.async_copy(
                    stg_u_hbm.at[uvec[i]], rows_u.at[k], sem_u))
                copies.append(pltpu.async_copy(
                    stg_v_hbm.at[ivec[i]], rows_v.at[k], sem_v))
        for c in copies:
            c.wait()
        for g in range(CHUNK // LANES):
            sums = jnp.zeros((LANES,), jnp.float32)
            for i in range(LANES):
                k = g * LANES + i
                s = rows_u[k, pl.ds(0, LANES)] * rows_v[k, pl.ds(0, LANES)]
                for c in range(1, EMBED_DIM // LANES):
                    u = rows_u[k, pl.ds(c * LANES, LANES)]
                    v = rows_v[k, pl.ds(c * LANES, LANES)]
                    s = s + u * v
                lane_mask = jnp.arange(LANES, dtype=jnp.int32) == i
                sums = jnp.where(lane_mask, jnp.sum(s), sums)
            out_vals[pl.ds(base + g * LANES, LANES)] = sums
        return carry

    lax.fori_loop(0, ROWS_PER_WORKER // CHUNK, chunk_body, 0)

    pltpu.sync_copy(out_vals, out_hbm.at[wid])


def _schedule(sorted_ids):
    pan = lax.shift_right_logical(sorted_ids, 7)
    first = jnp.concatenate(
        [jnp.ones((NUM_WORKERS, 1), jnp.bool_),
         pan[:, 1:] != pan[:, :-1]], axis=1)
    slot = jnp.cumsum(first.astype(jnp.int32), axis=1) - 1
    nslots = slot[:, -1] + 1
    widx = jnp.broadcast_to(
        jnp.arange(NUM_WORKERS, dtype=jnp.int32)[:, None],
        (NUM_WORKERS, QROWS))
    jidx = jnp.broadcast_to(
        jnp.arange(QROWS, dtype=jnp.int32)[None, :], (NUM_WORKERS, QROWS))
    psched = jnp.zeros((NUM_WORKERS, QROWS), jnp.int32) \
        .at[widx, slot].set(jnp.minimum(pan, NUM_PANELS - 1))
    start = jnp.full((NUM_WORKERS, ST_LEN), QROWS, jnp.int32) \
        .at[widx, slot].min(jidx)
    return psched, start, nslots.astype(jnp.int32)


def kernel(user_ids, item_ids, user_table, item_table):
    uid = user_ids.astype(jnp.int32)
    iid = item_ids.astype(jnp.int32)
    su = jnp.sort(uid)
    sv = jnp.sort(iid)
    rng = jnp.arange(BATCH, dtype=jnp.int32)
    inv_u = jnp.zeros((BATCH,), jnp.int32).at[jnp.argsort(uid)].set(rng)
    inv_v = jnp.zeros((BATCH,), jnp.int32).at[jnp.argsort(iid)].set(rng)
    utT = user_table.T
    itT = item_table.T
    utail = user_table[TAIL_BASE:]
    itail = item_table[TAIL_BASE:]

    stg_us = []
    stg_vs = []
    for q in range(NUM_Q):
        suq = lax.slice_in_dim(su, q * QBATCH, (q + 1) * QBATCH) \
            .reshape(NUM_WORKERS, QROWS)
        svq = lax.slice_in_dim(sv, q * QBATCH, (q + 1) * QBATCH) \
            .reshape(NUM_WORKERS, QROWS)
        ups, ust, uns = _schedule(suq)
        vps, vst, vns = _schedule(svq)
        sq_u, sq_v = _gather_kernel(
            suq, ups, ust, uns, svq, vps, vst, vns,
            utT, itT, utail, itail)
        stg_us.append(sq_u.reshape(QBATCH, EMBED_DIM))
        stg_vs.append(sq_v.reshape(QBATCH, EMBED_DIM))

    stg_u = jnp.concatenate(stg_us, axis=0)
    stg_v = jnp.concatenate(stg_vs, axis=0)
    out = _dot_kernel(
        inv_u.reshape(NUM_WORKERS, ROWS_PER_WORKER),
        inv_v.reshape(NUM_WORKERS, ROWS_PER_WORKER),
        stg_u, stg_v)
    return out.reshape(BATCH)


# speedup vs baseline: 1.3559x; 1.3559x over previous
import functools

import jax
import jax.numpy as jnp
from jax import lax
from jax.experimental import pallas as pl
from jax.experimental.pallas import tpu as pltpu
from jax.experimental.pallas import tpu_sc as plsc

NUM_USERS = 1000000
BATCH = 16384
EMBED_DIM = 64
PANEL_W = 128
SLAB_W = 512
NUM_SLABS = NUM_USERS // SLAB_W
TAIL_BASE = NUM_SLABS * SLAB_W
TAIL_W = NUM_USERS - TAIL_BASE
NUM_CORES = 2
NUM_SUBCORES = 16
NUM_WORKERS = NUM_CORES * NUM_SUBCORES
ROWS_PER_WORKER = BATCH // NUM_WORKERS
NUM_Q = 4
QBATCH = BATCH // NUM_Q
QROWS = QBATCH // NUM_WORKERS
LANES = 16
ST_LEN = QROWS + LANES
GROUPS = ROWS_PER_WORKER // LANES
CHUNK = 32

_mesh = plsc.VectorSubcoreMesh(core_axis_name="c", subcore_axis_name="s")


def _sread(ref, j):
    base = lax.shift_right_logical(j, 4) * LANES
    v = ref[pl.ds(base, LANES)]
    lane = jnp.full((LANES,), j & (LANES - 1), jnp.int32)
    return v.at[lane].get(mode="promise_in_bounds")[0]


@functools.partial(
    pl.kernel,
    out_type=(
        jax.ShapeDtypeStruct((NUM_WORKERS, QROWS, EMBED_DIM), jnp.float32),
        jax.ShapeDtypeStruct((NUM_WORKERS, QROWS, EMBED_DIM), jnp.float32),
    ),
    mesh=_mesh,
    compiler_params=pltpu.CompilerParams(needs_layout_passes=False),
    scratch_types=[
        pltpu.VMEM((QROWS,), jnp.int32),
        pltpu.VMEM((QROWS,), jnp.int32),
        pltpu.VMEM((ST_LEN,), jnp.int32),
        pltpu.VMEM((2 * LANES,), jnp.int32),
        pltpu.VMEM((EMBED_DIM, SLAB_W), jnp.float32),
        pltpu.VMEM((TAIL_W, EMBED_DIM), jnp.float32),
        pltpu.VMEM((QROWS, EMBED_DIM), jnp.float32),
    ],
)
def _gather_kernel(su_hbm, ups_hbm, ust_hbm, uns_hbm,
                   sv_hbm, vps_hbm, vst_hbm, vns_hbm,
                   utT_hbm, itT_hbm, utail_hbm, itail_hbm,
                   stg_u_hbm, stg_v_hbm,
                   sid_v, ps_v, st_v, ns_v, panel, tailb, bigbuf):
    wid = lax.axis_index("s") * NUM_CORES + lax.axis_index("c")

    def do_table(ids_hbm, ps_hbm, sta_hbm, ns_hbm, tbl_hbm, tail_hbm,
                 stg_hbm):
        pltpu.sync_copy(ids_hbm.at[wid], sid_v)
        pltpu.sync_copy(ps_hbm.at[wid], ps_v)
        pltpu.sync_copy(sta_hbm.at[wid], st_v)
        pltpu.sync_copy(ns_hbm, ns_v)
        pltpu.sync_copy(tail_hbm, tailb)
        nslots = _sread(ns_v, wid)

        def id_body(j, c2):
            id_ = _sread(sid_v, j)
            is_tail = id_ >= TAIL_BASE
            col = jnp.full((LANES,), id_ & (SLAB_W - 1), jnp.int32)
            tr = jnp.full(
                (LANES,), jnp.where(is_tail, id_ - TAIL_BASE, 0), jnp.int32)
            for c in range(EMBED_DIM // LANES):
                dvec = jnp.full((LANES,), c * LANES, jnp.int32) \
                    + lax.iota(jnp.int32, LANES)
                u_p = plsc.load_gather(panel, [dvec, col])
                u_t = plsc.load_gather(tailb, [tr, dvec])
                bigbuf[j, pl.ds(c * LANES, LANES)] = \
                    jnp.where(is_tail, u_t, u_p)
            return c2

        def slot_body(s, carry):
            pnl = _sread(ps_v, s)
            start = _sread(st_v, s)
            end = _sread(st_v, s + 1)
            off = pl.multiple_of(pnl * SLAB_W, SLAB_W)
            pltpu.sync_copy(tbl_hbm.at[:, pl.ds(off, SLAB_W)], panel)
            lax.fori_loop(start, end, id_body, 0)
            return carry

        lax.fori_loop(0, nslots, slot_body, 0)
        pltpu.sync_copy(bigbuf, stg_hbm.at[wid])

    do_table(su_hbm, ups_hbm, ust_hbm, uns_hbm, utT_hbm, utail_hbm,
             stg_u_hbm)
    do_table(sv_hbm, vps_hbm, vst_hbm, vns_hbm, itT_hbm, itail_hbm,
             stg_v_hbm)


@functools.partial(
    pl.kernel,
    out_type=jax.ShapeDtypeStruct((NUM_WORKERS, ROWS_PER_WORKER), jnp.float32),
    mesh=_mesh,
    compiler_params=pltpu.CompilerParams(needs_layout_passes=False),
    scratch_types=[
        pltpu.VMEM((ROWS_PER_WORKER,), jnp.int32),
        pltpu.VMEM((ROWS_PER_WORKER,), jnp.int32),
        pltpu.VMEM((CHUNK, EMBED_DIM), jnp.float32),
        pltpu.VMEM((CHUNK, EMBED_DIM), jnp.float32),
        pltpu.VMEM((ROWS_PER_WORKER,), jnp.float32),
        pltpu.SemaphoreType.DMA,
        pltpu.SemaphoreType.DMA,
    ],
)
def _dot_kernel(ui_hbm, vi_hbm, stg_u_hbm, stg_v_hbm, out_hbm,
                ui_v, vi_v, rows_u, rows_v, out_vals, sem_u, sem_v):
    wid = lax.axis_index("s") * NUM_CORES + lax.axis_index("c")

    pltpu.sync_copy(ui_hbm.at[wid], ui_v)
    pltpu.sync_copy(vi_hbm.at[wid], vi_v)

    def chunk_body(ch, carry):
        base = ch * CHUNK
        copies = []
        for g in range(CHUNK // LANES):
            uvec = ui_v[pl.ds(base + g * LANES, LANES)]
            ivec = vi_v[pl.ds(base + g * LANES, LANES)]
            for i in range(LANES):
                k = g * LANES + i
                copies.append(pltpu.async_copy(
                    stg_u_hbm.at[uvec[i]], rows_u.at[k], sem_u))
                copies.append(pltpu.async_copy(
                    stg_v_hbm.at[ivec[i]], rows_v.at[k], sem_v))
        for c in copies:
            c.wait()
        for g in range(CHUNK // LANES):
            sums = jnp.zeros((LANES,), jnp.float32)
            for i in range(LANES):
                k = g * LANES + i
                s = rows_u[k, pl.ds(0, LANES)] * rows_v[k, pl.ds(0, LANES)]
                for c in range(1, EMBED_DIM // LANES):
                    u = rows_u[k, pl.ds(c * LANES, LANES)]
                    v = rows_v[k, pl.ds(c * LANES, LANES)]
                    s = s + u * v
                lane_mask = jnp.arange(LANES, dtype=jnp.int32) == i
                sums = jnp.where(lane_mask, jnp.sum(s), sums)
            out_vals[pl.ds(base + g * LANES, LANES)] = sums
        return carry

    lax.fori_loop(0, ROWS_PER_WORKER // CHUNK, chunk_body, 0)

    pltpu.sync_copy(out_vals, out_hbm.at[wid])


def _schedule(sorted_ids):
    pan = lax.shift_right_logical(sorted_ids, 9)
    first = jnp.concatenate(
        [jnp.ones((NUM_WORKERS, 1), jnp.bool_),
         pan[:, 1:] != pan[:, :-1]], axis=1)
    slot = jnp.cumsum(first.astype(jnp.int32), axis=1) - 1
    nslots = slot[:, -1] + 1
    widx = jnp.broadcast_to(
        jnp.arange(NUM_WORKERS, dtype=jnp.int32)[:, None],
        (NUM_WORKERS, QROWS))
    jidx = jnp.broadcast_to(
        jnp.arange(QROWS, dtype=jnp.int32)[None, :], (NUM_WORKERS, QROWS))
    psched = jnp.zeros((NUM_WORKERS, QROWS), jnp.int32) \
        .at[widx, slot].set(jnp.minimum(pan, NUM_SLABS - 1))
    start = jnp.full((NUM_WORKERS, ST_LEN), QROWS, jnp.int32) \
        .at[widx, slot].min(jidx)
    return psched, start, nslots.astype(jnp.int32)


def kernel(user_ids, item_ids, user_table, item_table):
    uid = user_ids.astype(jnp.int32)
    iid = item_ids.astype(jnp.int32)
    su = jnp.sort(uid)
    sv = jnp.sort(iid)
    rng = jnp.arange(BATCH, dtype=jnp.int32)
    inv_u = jnp.zeros((BATCH,), jnp.int32).at[jnp.argsort(uid)].set(rng)
    inv_v = jnp.zeros((BATCH,), jnp.int32).at[jnp.argsort(iid)].set(rng)
    utT = user_table.T
    itT = item_table.T
    utail = user_table[TAIL_BASE:]
    itail = item_table[TAIL_BASE:]

    stg_us = []
    stg_vs = []
    for q in range(NUM_Q):
        suq = lax.slice_in_dim(su, q * QBATCH, (q + 1) * QBATCH) \
            .reshape(NUM_WORKERS, QROWS)
        svq = lax.slice_in_dim(sv, q * QBATCH, (q + 1) * QBATCH) \
            .reshape(NUM_WORKERS, QROWS)
        ups, ust, uns = _schedule(suq)
        vps, vst, vns = _schedule(svq)
        sq_u, sq_v = _gather_kernel(
            suq, ups, ust, uns, svq, vps, vst, vns,
            utT, itT, utail, itail)
        stg_us.append(sq_u.reshape(QBATCH, EMBED_DIM))
        stg_vs.append(sq_v.reshape(QBATCH, EMBED_DIM))

    stg_u = jnp.concatenate(stg_us, axis=0)
    stg_v = jnp.concatenate(stg_vs, axis=0)
    out = _dot_kernel(
        inv_u.reshape(NUM_WORKERS, ROWS_PER_WORKER),
        inv_v.reshape(NUM_WORKERS, ROWS_PER_WORKER),
        stg_u, stg_v)
    return out.reshape(BATCH)
